# Initial kernel scaffold; baseline (speedup 1.0000x reference)
import jax
import jax.numpy as jnp
from jax.experimental import pallas as pl


def kernel(x, context, gn_gamma, gn_beta, w_in, b_in, w_out, b_out, g1, b1, g2, b2, g3, b3, a1_w_qkv, a1_w_q_scaled, a1_w_kv, a1_w_o, a1_b_o, a2_w_q_scaled, a2_w_kv, a2_w_o, a2_b_o, ff_w_x, ff_w_g, ff_b_x, ff_b_g, ff_w_o, ff_b_o):
    raise NotImplementedError("write your pallas kernel here")


# speedup vs baseline: 2.3867x; 2.3867x over previous
import functools

import jax
import jax.numpy as jnp
from jax.experimental import pallas as pl
from jax.experimental.pallas import tpu as pltpu

_VMEM_LIMIT = 64 * 1024 * 1024
_BF = jnp.bfloat16
_F32 = jnp.float32


def _ln(x, g, b, eps=1e-5):
    mu = jnp.mean(x, axis=-1, keepdims=True)
    xc = x - mu
    var = jnp.mean(xc * xc, axis=-1, keepdims=True)
    return (xc * jax.lax.rsqrt(var + eps)) * g + b


def _mha(q_loader, k_loader, v_loader, heads, dh):
    outs = []
    for h in range(heads):
        lo = h * dh
        qh = q_loader(lo)
        kh = k_loader(lo)
        vh = v_loader(lo)
        s = jax.lax.dot_general(qh, kh, (((1,), (1,)), ((), ())),
                                preferred_element_type=_F32)
        m = jnp.max(s, axis=-1, keepdims=True)
        p = jnp.exp(s - m)
        l = jnp.sum(p, axis=-1, keepdims=True)
        o = jnp.dot(p.astype(_BF), vh, preferred_element_type=_F32)
        outs.append(o / l)
    return jnp.concatenate(outs, axis=-1).astype(_BF)


def _pre_kernel(x_ref, gng_ref, beff_ref, w_in_ref, g1_ref, b1_ref, wqkv_ref,
                ctx_ref, wkv2_ref, hres_ref, qkv_ref, kv2_ref, *, groups):
    xg = x_ref[0].astype(_F32)
    C, HW = xg.shape
    xr = xg.reshape(groups, (C // groups) * HW)
    mu = jnp.mean(xr, axis=-1, keepdims=True)
    xc = xr - mu
    var = jnp.mean(xc * xc, axis=-1, keepdims=True)
    xn = (xc * jax.lax.rsqrt(var + 1e-6)).reshape(C, HW)
    xs = (xn * gng_ref[...].astype(_F32)).astype(_BF)
    h = jax.lax.dot_general(xs, w_in_ref[...], (((0,), (0,)), ((), ())),
                            preferred_element_type=_F32)
    h = h + beff_ref[...]
    hres_ref[0] = h
    hn = _ln(h, g1_ref[...], b1_ref[...])
    qkv_ref[0] = jnp.dot(hn.astype(_BF), wqkv_ref[...],
                         preferred_element_type=_F32).astype(_BF)
    kv2_ref[0] = jnp.dot(ctx_ref[0], wkv2_ref[...],
                         preferred_element_type=_F32).astype(_BF)


def _attn1_kernel(qkv_ref, wo_ref, bo_ref, res_ref, g2_ref, b2_ref, wq2_ref,
                  x2_ref, q2_ref, *, heads, dh, tq):
    qi = pl.program_id(1)
    inner = heads * dh
    base = qi * tq
    attn = _mha(
        lambda lo: qkv_ref[0, pl.ds(base, tq), lo:lo + dh],
        lambda lo: qkv_ref[0, :, inner + lo:inner + lo + dh],
        lambda lo: qkv_ref[0, :, 2 * inner + lo:2 * inner + lo + dh],
        heads, dh)
    x2 = jnp.dot(attn, wo_ref[...], preferred_element_type=_F32)
    x2 = x2 + bo_ref[...] + res_ref[0]
    x2_ref[0] = x2
    hn = _ln(x2, g2_ref[...], b2_ref[...])
    q2_ref[0] = jnp.dot(hn.astype(_BF), wq2_ref[...],
                        preferred_element_type=_F32).astype(_BF)


def _attn2_ff_kernel(q2_ref, kv2_ref, wo2_ref, bo2_ref, res_ref, g3_ref, b3_ref,
                     wx_ref, wg_ref, bx_ref, bg_ref, wfo_ref, bfo_ref,
                     wout_ref, bout_ref, xin_ref, out_ref, *, heads, dh):
    inner = heads * dh
    attn = _mha(
        lambda lo: q2_ref[0, :, lo:lo + dh],
        lambda lo: kv2_ref[0, :, lo:lo + dh],
        lambda lo: kv2_ref[0, :, inner + lo:inner + lo + dh],
        heads, dh)
    x3 = jnp.dot(attn, wo2_ref[...], preferred_element_type=_F32)
    x3 = x3 + bo2_ref[...] + res_ref[0]
    hn = _ln(x3, g3_ref[...], b3_ref[...]).astype(_BF)
    u = jnp.dot(hn, wx_ref[...], preferred_element_type=_F32) + bx_ref[...]
    g = jnp.dot(hn, wg_ref[...], preferred_element_type=_F32) + bg_ref[...]
    gg = (u * jax.nn.gelu(g)).astype(_BF)
    x4 = jnp.dot(gg, wfo_ref[...], preferred_element_type=_F32)
    x4 = x4 + bfo_ref[...] + x3
    yt = jax.lax.dot_general(wout_ref[...], x4.astype(_BF),
                             (((0,), (1,)), ((), ())),
                             preferred_element_type=_F32)
    out_ref[0] = yt + bout_ref[...] + xin_ref[0].astype(_F32)


def kernel(x, context, gn_gamma, gn_beta, w_in, b_in, w_out, b_out,
           g1, b1, g2, b2, g3, b3,
           a1_w_qkv, a1_w_q_scaled, a1_w_kv, a1_w_o, a1_b_o,
           a2_w_q_scaled, a2_w_kv, a2_w_o, a2_b_o,
           ff_w_x, ff_w_g, ff_b_x, ff_b_g, ff_w_o, ff_b_o):
    B, C, H, W = x.shape
    HW = H * W
    heads, dh = 8, 40
    inner = heads * dh
    Lc = context.shape[1]
    dff = ff_w_x.shape[1]
    tq = min(256, HW)
    QT = HW // tq

    x3d = x.reshape(B, C, HW)
    beff = (b_in + gn_beta @ w_in).reshape(1, inner)
    row = lambda v: v.reshape(1, -1)
    col = lambda v: v.reshape(-1, 1)

    sem = pltpu.CompilerParams(
        dimension_semantics=("parallel",), vmem_limit_bytes=_VMEM_LIMIT)
    sem2 = pltpu.CompilerParams(
        dimension_semantics=("parallel", "arbitrary"),
        vmem_limit_bytes=_VMEM_LIMIT)

    full = lambda *shape: pl.BlockSpec(shape, lambda b, qi=0: (0,) * len(shape))

    hres, qkv, kv2 = pl.pallas_call(
        functools.partial(_pre_kernel, groups=32),
        grid=(B,),
        in_specs=[
            pl.BlockSpec((1, C, HW), lambda b: (b, 0, 0)),
            pl.BlockSpec((C, 1), lambda b: (0, 0)),
            pl.BlockSpec((1, inner), lambda b: (0, 0)),
            pl.BlockSpec((C, inner), lambda b: (0, 0)),
            pl.BlockSpec((1, inner), lambda b: (0, 0)),
            pl.BlockSpec((1, inner), lambda b: (0, 0)),
            pl.BlockSpec((inner, 3 * inner), lambda b: (0, 0)),
            pl.BlockSpec((1, Lc, context.shape[2]), lambda b: (b, 0, 0)),
            pl.BlockSpec((context.shape[2], 2 * inner), lambda b: (0, 0)),
        ],
        out_specs=[
            pl.BlockSpec((1, HW, inner), lambda b: (b, 0, 0)),
            pl.BlockSpec((1, HW, 3 * inner), lambda b: (b, 0, 0)),
            pl.BlockSpec((1, Lc, 2 * inner), lambda b: (b, 0, 0)),
        ],
        out_shape=[
            jax.ShapeDtypeStruct((B, HW, inner), _F32),
            jax.ShapeDtypeStruct((B, HW, 3 * inner), _BF),
            jax.ShapeDtypeStruct((B, Lc, 2 * inner), _BF),
        ],
        compiler_params=sem,
    )(x3d, col(gn_gamma), beff, w_in.astype(_BF), row(g1), row(b1),
      a1_w_qkv.astype(_BF), context.astype(_BF), a2_w_kv.astype(_BF))

    x2res, q2 = pl.pallas_call(
        functools.partial(_attn1_kernel, heads=heads, dh=dh, tq=tq),
        grid=(B, QT),
        in_specs=[
            pl.BlockSpec((1, HW, 3 * inner), lambda b, qi: (b, 0, 0)),
            full(inner, inner),
            full(1, inner),
            pl.BlockSpec((1, tq, inner), lambda b, qi: (b, qi, 0)),
            full(1, inner),
            full(1, inner),
            full(inner, inner),
        ],
        out_specs=[
            pl.BlockSpec((1, tq, inner), lambda b, qi: (b, qi, 0)),
            pl.BlockSpec((1, tq, inner), lambda b, qi: (b, qi, 0)),
        ],
        out_shape=[
            jax.ShapeDtypeStruct((B, HW, inner), _F32),
            jax.ShapeDtypeStruct((B, HW, inner), _BF),
        ],
        compiler_params=sem2,
    )(qkv, a1_w_o.astype(_BF), row(a1_b_o), hres, row(g2), row(b2),
      a2_w_q_scaled.astype(_BF))

    out = pl.pallas_call(
        functools.partial(_attn2_ff_kernel, heads=heads, dh=dh),
        grid=(B, QT),
        in_specs=[
            pl.BlockSpec((1, tq, inner), lambda b, qi: (b, qi, 0)),
            pl.BlockSpec((1, Lc, 2 * inner), lambda b, qi: (b, 0, 0)),
            full(inner, inner),
            full(1, inner),
            pl.BlockSpec((1, tq, inner), lambda b, qi: (b, qi, 0)),
            full(1, inner),
            full(1, inner),
            full(inner, dff),
            full(inner, dff),
            full(1, dff),
            full(1, dff),
            full(dff, inner),
            full(1, inner),
            full(inner, C),
            full(C, 1),
            pl.BlockSpec((1, C, tq), lambda b, qi: (b, 0, qi)),
        ],
        out_specs=pl.BlockSpec((1, C, tq), lambda b, qi: (b, 0, qi)),
        out_shape=jax.ShapeDtypeStruct((B, C, HW), _F32),
        compiler_params=sem2,
    )(q2, kv2, a2_w_o.astype(_BF), row(a2_b_o), x2res, row(g3), row(b3),
      ff_w_x.astype(_BF), ff_w_g.astype(_BF), row(ff_b_x), row(ff_b_g),
      ff_w_o.astype(_BF), row(ff_b_o), w_out.astype(_BF), col(b_out), x3d)

    return out.reshape(B, C, H, W)


# speedup vs baseline: 2.4289x; 1.0177x over previous
import functools

import jax
import jax.numpy as jnp
from jax.experimental import pallas as pl
from jax.experimental.pallas import tpu as pltpu

_VMEM_LIMIT = 64 * 1024 * 1024
_BF = jnp.bfloat16
_F32 = jnp.float32


def _ln_cm(x, g, b, eps=1e-5):
    mu = jnp.mean(x, axis=0, keepdims=True)
    xc = x - mu
    var = jnp.mean(xc * xc, axis=0, keepdims=True)
    return (xc * jax.lax.rsqrt(var + eps)) * g + b


def _mha_cm(q_loader, k_loader, v_loader, heads, dh):
    outs = []
    for h in range(heads):
        lo = h * dh
        qh = q_loader(lo)
        kh = k_loader(lo)
        vh = v_loader(lo)
        st = jax.lax.dot_general(kh, qh, (((0,), (0,)), ((), ())),
                                 preferred_element_type=_F32)
        m = jnp.max(st, axis=0, keepdims=True)
        p = jnp.exp(st - m)
        l = jnp.sum(p, axis=0, keepdims=True)
        ot = jnp.dot(vh, p.astype(_BF), preferred_element_type=_F32)
        outs.append(ot * (1.0 / l))
    return jnp.concatenate(outs, axis=0).astype(_BF)


def _pre_kernel(x_ref, gng_ref, beff_ref, w_in_t_ref, g1_ref, b1_ref,
                wq_t_ref, wkv_t_ref, ctx_ref, wkv2_t_ref,
                hres_ref, qt_ref, kvt_ref, kv2t_ref, *, groups):
    xg = x_ref[0].astype(_F32)
    C, HW = xg.shape
    xr = xg.reshape(groups, (C // groups) * HW)
    mu = jnp.mean(xr, axis=-1, keepdims=True)
    xc = xr - mu
    var = jnp.mean(xc * xc, axis=-1, keepdims=True)
    xn = (xc * jax.lax.rsqrt(var + 1e-6)).reshape(C, HW)
    xs = (xn * gng_ref[...].astype(_F32)).astype(_BF)
    h = jnp.dot(w_in_t_ref[...], xs, preferred_element_type=_F32)
    h = h + beff_ref[...]
    hres_ref[0] = h
    hn = _ln_cm(h, g1_ref[...], b1_ref[...]).astype(_BF)
    qt_ref[0] = jnp.dot(wq_t_ref[...], hn,
                        preferred_element_type=_F32).astype(_BF)
    kvt_ref[0] = jnp.dot(wkv_t_ref[...], hn,
                         preferred_element_type=_F32).astype(_BF)
    kv2t_ref[0] = jax.lax.dot_general(wkv2_t_ref[...], ctx_ref[0],
                                      (((1,), (1,)), ((), ())),
                                      preferred_element_type=_F32).astype(_BF)


def _attn1_kernel(qt_ref, kvt_ref, wo_t_ref, bo_ref, res_ref, g2_ref, b2_ref,
                  wq2_t_ref, x2_ref, q2_ref, *, heads, dh):
    inner = heads * dh
    attn = _mha_cm(
        lambda lo: qt_ref[0, lo:lo + dh, :],
        lambda lo: kvt_ref[0, lo:lo + dh, :],
        lambda lo: kvt_ref[0, inner + lo:inner + lo + dh, :],
        heads, dh)
    x2 = jnp.dot(wo_t_ref[...], attn, preferred_element_type=_F32)
    x2 = x2 + bo_ref[...] + res_ref[0]
    x2_ref[0] = x2
    hn = _ln_cm(x2, g2_ref[...], b2_ref[...]).astype(_BF)
    q2_ref[0] = jnp.dot(wq2_t_ref[...], hn,
                        preferred_element_type=_F32).astype(_BF)


def _attn2_ff_kernel(q2_ref, kv2t_ref, wo2_t_ref, bo2_ref, res_ref,
                     g3_ref, b3_ref, wx_t_ref, wg_t_ref, bx_ref, bg_ref,
                     wfo_t_ref, bfo_ref, wout_t_ref, bout_ref, xin_ref,
                     out_ref, *, heads, dh):
    inner = heads * dh
    attn = _mha_cm(
        lambda lo: q2_ref[0, lo:lo + dh, :],
        lambda lo: kv2t_ref[0, lo:lo + dh, :],
        lambda lo: kv2t_ref[0, inner + lo:inner + lo + dh, :],
        heads, dh)
    x3 = jnp.dot(wo2_t_ref[...], attn, preferred_element_type=_F32)
    x3 = x3 + bo2_ref[...] + res_ref[0]
    hn = _ln_cm(x3, g3_ref[...], b3_ref[...]).astype(_BF)
    u = jnp.dot(wx_t_ref[...], hn, preferred_element_type=_F32) + bx_ref[...]
    g = jnp.dot(wg_t_ref[...], hn, preferred_element_type=_F32) + bg_ref[...]
    gg = (u * jax.nn.gelu(g)).astype(_BF)
    x4 = jnp.dot(wfo_t_ref[...], gg, preferred_element_type=_F32)
    x4 = x4 + bfo_ref[...] + x3
    yt = jnp.dot(wout_t_ref[...], x4.astype(_BF),
                 preferred_element_type=_F32)
    out_ref[0] = yt + bout_ref[...] + xin_ref[0].astype(_F32)


def kernel(x, context, gn_gamma, gn_beta, w_in, b_in, w_out, b_out,
           g1, b1, g2, b2, g3, b3,
           a1_w_qkv, a1_w_q_scaled, a1_w_kv, a1_w_o, a1_b_o,
           a2_w_q_scaled, a2_w_kv, a2_w_o, a2_b_o,
           ff_w_x, ff_w_g, ff_b_x, ff_b_g, ff_w_o, ff_b_o):
    B, C, H, W = x.shape
    HW = H * W
    heads, dh = 8, 40
    inner = heads * dh
    Lc = context.shape[1]
    Dc = context.shape[2]
    dff = ff_w_x.shape[1]
    tq = min(256, HW)
    QT = HW // tq

    x3d = x.reshape(B, C, HW)
    beff = (b_in + gn_beta @ w_in).reshape(-1, 1)
    col = lambda v: v.reshape(-1, 1)
    bt = lambda w: w.T.astype(_BF)
    qkv_t = a1_w_qkv.T.astype(_BF)

    sem = pltpu.CompilerParams(
        dimension_semantics=("parallel",), vmem_limit_bytes=_VMEM_LIMIT)
    sem2 = pltpu.CompilerParams(
        dimension_semantics=("parallel", "arbitrary"),
        vmem_limit_bytes=_VMEM_LIMIT)

    full = lambda *shape: pl.BlockSpec(shape, lambda b, qi=0: (0,) * len(shape))

    hres, qt, kvt, kv2t = pl.pallas_call(
        functools.partial(_pre_kernel, groups=32),
        grid=(B,),
        in_specs=[
            pl.BlockSpec((1, C, HW), lambda b: (b, 0, 0)),
            full(C, 1), full(inner, 1), full(inner, C),
            full(inner, 1), full(inner, 1),
            full(inner, inner), full(2 * inner, inner),
            pl.BlockSpec((1, Lc, Dc), lambda b: (b, 0, 0)),
            full(2 * inner, Dc),
        ],
        out_specs=[
            pl.BlockSpec((1, inner, HW), lambda b: (b, 0, 0)),
            pl.BlockSpec((1, inner, HW), lambda b: (b, 0, 0)),
            pl.BlockSpec((1, 2 * inner, HW), lambda b: (b, 0, 0)),
            pl.BlockSpec((1, 2 * inner, Lc), lambda b: (b, 0, 0)),
        ],
        out_shape=[
            jax.ShapeDtypeStruct((B, inner, HW), _F32),
            jax.ShapeDtypeStruct((B, inner, HW), _BF),
            jax.ShapeDtypeStruct((B, 2 * inner, HW), _BF),
            jax.ShapeDtypeStruct((B, 2 * inner, Lc), _BF),
        ],
        compiler_params=sem,
    )(x3d, col(gn_gamma), beff, bt(w_in), col(g1), col(b1),
      qkv_t[:inner], qkv_t[inner:], context.astype(_BF), bt(a2_w_kv))

    tile = pl.BlockSpec((1, inner, tq), lambda b, qi: (b, 0, qi))
    x2res, q2 = pl.pallas_call(
        functools.partial(_attn1_kernel, heads=heads, dh=dh),
        grid=(B, QT),
        in_specs=[
            tile,
            pl.BlockSpec((1, 2 * inner, HW), lambda b, qi: (b, 0, 0)),
            full(inner, inner), full(inner, 1),
            tile,
            full(inner, 1), full(inner, 1),
            full(inner, inner),
        ],
        out_specs=[tile, tile],
        out_shape=[
            jax.ShapeDtypeStruct((B, inner, HW), _F32),
            jax.ShapeDtypeStruct((B, inner, HW), _BF),
        ],
        compiler_params=sem2,
    )(qt, kvt, bt(a1_w_o), col(a1_b_o), hres, col(g2), col(b2),
      bt(a2_w_q_scaled))

    out = pl.pallas_call(
        functools.partial(_attn2_ff_kernel, heads=heads, dh=dh),
        grid=(B, QT),
        in_specs=[
            tile,
            pl.BlockSpec((1, 2 * inner, Lc), lambda b, qi: (b, 0, 0)),
            full(inner, inner), full(inner, 1),
            tile,
            full(inner, 1), full(inner, 1),
            full(dff, inner), full(dff, inner), full(dff, 1), full(dff, 1),
            full(inner, dff), full(inner, 1),
            full(C, inner), full(C, 1),
            pl.BlockSpec((1, C, tq), lambda b, qi: (b, 0, qi)),
        ],
        out_specs=pl.BlockSpec((1, C, tq), lambda b, qi: (b, 0, qi)),
        out_shape=jax.ShapeDtypeStruct((B, C, HW), _F32),
        compiler_params=sem2,
    )(q2, kv2t, bt(a2_w_o), col(a2_b_o), x2res, col(g3), col(b3),
      bt(ff_w_x), bt(ff_w_g), col(ff_b_x), col(ff_b_g),
      bt(ff_w_o), col(ff_b_o), bt(w_out), col(b_out), x3d)

    return out.reshape(B, C, H, W)


# speedup vs baseline: 3.2870x; 1.3532x over previous
import functools

import jax
import jax.numpy as jnp
from jax.experimental import pallas as pl
from jax.experimental.pallas import tpu as pltpu

_VMEM_LIMIT = 64 * 1024 * 1024
_BF = jnp.bfloat16
_F32 = jnp.float32


def _ln_cm(x, g, b, eps=1e-5):
    mu = jnp.mean(x, axis=0, keepdims=True)
    xc = x - mu
    var = jnp.mean(xc * xc, axis=0, keepdims=True)
    return (xc * jax.lax.rsqrt(var + eps)) * g + b


def _mha_cm(q_loader, k_loader, v_loader, heads, dh):
    outs = []
    for h in range(heads):
        lo = h * dh
        qh = q_loader(lo)
        kh = k_loader(lo)
        vh = v_loader(lo)
        st = jax.lax.dot_general(kh, qh, (((0,), (0,)), ((), ())),
                                 preferred_element_type=_F32)
        m = jnp.max(st, axis=0, keepdims=True)
        p = jnp.exp(st - m)
        l = jnp.sum(p, axis=0, keepdims=True)
        ot = jnp.dot(vh, p.astype(_BF), preferred_element_type=_F32)
        outs.append(ot * (1.0 / l))
    return jnp.concatenate(outs, axis=0).astype(_BF)


def _pre_kernel(x_ref, gng_ref, beff_ref, w_in_t_ref, g1_ref, b1_ref,
                wq_t_ref, wkv_t_ref, ctx_ref, wkv2_t_ref,
                hres_ref, qt_ref, kvt_ref, kv2t_ref, *, groups):
    xg = x_ref[0].astype(_F32)
    C, HW = xg.shape
    xr = xg.reshape(groups, (C // groups) * HW)
    mu = jnp.mean(xr, axis=-1, keepdims=True)
    xc = xr - mu
    var = jnp.mean(xc * xc, axis=-1, keepdims=True)
    xn = (xc * jax.lax.rsqrt(var + 1e-6)).reshape(C, HW)
    xs = (xn * gng_ref[...].astype(_F32)).astype(_BF)
    h = jnp.dot(w_in_t_ref[...], xs, preferred_element_type=_F32)
    h = h + beff_ref[...]
    hres_ref[0] = h
    hn = _ln_cm(h, g1_ref[...], b1_ref[...]).astype(_BF)
    qt_ref[0] = jnp.dot(wq_t_ref[...], hn,
                        preferred_element_type=_F32).astype(_BF)
    kvt_ref[0] = jnp.dot(wkv_t_ref[...], hn,
                         preferred_element_type=_F32).astype(_BF)
    kv2t_ref[0] = jax.lax.dot_general(wkv2_t_ref[...], ctx_ref[0],
                                      (((1,), (1,)), ((), ())),
                                      preferred_element_type=_F32).astype(_BF)


def _attn1_kernel(qt_ref, kvt_ref, wo_t_ref, bo_ref, res_ref, g2_ref, b2_ref,
                  wq2_t_ref, x2_ref, q2_ref, *, heads, dh):
    inner = heads * dh
    attn = _mha_cm(
        lambda lo: qt_ref[0, lo:lo + dh, :],
        lambda lo: kvt_ref[0, lo:lo + dh, :],
        lambda lo: kvt_ref[0, inner + lo:inner + lo + dh, :],
        heads, dh)
    x2 = jnp.dot(wo_t_ref[...], attn, preferred_element_type=_F32)
    x2 = x2 + bo_ref[...] + res_ref[0]
    x2_ref[0] = x2
    hn = _ln_cm(x2, g2_ref[...], b2_ref[...]).astype(_BF)
    q2_ref[0] = jnp.dot(wq2_t_ref[...], hn,
                        preferred_element_type=_F32).astype(_BF)


def _attn2_ff_kernel(q2_ref, kv2t_ref, wo2_t_ref, bo2_ref, res_ref,
                     g3_ref, b3_ref, wx_t_ref, wg_t_ref, bx_ref, bg_ref,
                     wfo_t_ref, bfo_ref, wout_t_ref, bout_ref, xin_ref,
                     out_ref, *, heads, dh):
    inner = heads * dh
    attn = _mha_cm(
        lambda lo: q2_ref[0, lo:lo + dh, :],
        lambda lo: kv2t_ref[0, lo:lo + dh, :],
        lambda lo: kv2t_ref[0, inner + lo:inner + lo + dh, :],
        heads, dh)
    x3 = jnp.dot(wo2_t_ref[...], attn, preferred_element_type=_F32)
    x3 = x3 + bo2_ref[...] + res_ref[0]
    hn = _ln_cm(x3, g3_ref[...], b3_ref[...]).astype(_BF)
    u = jnp.dot(wx_t_ref[...], hn, preferred_element_type=_F32) + bx_ref[...]
    g = jnp.dot(wg_t_ref[...], hn, preferred_element_type=_F32) + bg_ref[...]
    gg = (u * jax.nn.gelu(g)).astype(_BF)
    x4 = jnp.dot(wfo_t_ref[...], gg, preferred_element_type=_F32)
    x4 = x4 + bfo_ref[...] + x3
    yt = jnp.dot(wout_t_ref[...], x4.astype(_BF),
                 preferred_element_type=_F32)
    out_ref[0] = yt + bout_ref[...] + xin_ref[0].astype(_F32)


def kernel(x, context, gn_gamma, gn_beta, w_in, b_in, w_out, b_out,
           g1, b1, g2, b2, g3, b3,
           a1_w_qkv, a1_w_q_scaled, a1_w_kv, a1_w_o, a1_b_o,
           a2_w_q_scaled, a2_w_kv, a2_w_o, a2_b_o,
           ff_w_x, ff_w_g, ff_b_x, ff_b_g, ff_w_o, ff_b_o):
    B, C, H, W = x.shape
    HW = H * W
    heads, dh = 8, 40
    inner = heads * dh
    Lc = context.shape[1]
    Dc = context.shape[2]
    dff = ff_w_x.shape[1]
    tq = min(512, HW)
    QT = HW // tq

    x3d = x.reshape(B, C, HW)
    beff = (b_in + gn_beta @ w_in).reshape(-1, 1)
    col = lambda v: v.reshape(-1, 1)
    bt = lambda w: w.T.astype(_BF)
    qkv_t = a1_w_qkv.T.astype(_BF)

    sem = pltpu.CompilerParams(
        dimension_semantics=("parallel",), vmem_limit_bytes=_VMEM_LIMIT)
    sem2 = pltpu.CompilerParams(
        dimension_semantics=("parallel", "arbitrary"),
        vmem_limit_bytes=_VMEM_LIMIT)

    full = lambda *shape: pl.BlockSpec(shape, lambda b, qi=0: (0,) * len(shape))

    hres, qt, kvt, kv2t = pl.pallas_call(
        functools.partial(_pre_kernel, groups=32),
        grid=(B,),
        in_specs=[
            pl.BlockSpec((1, C, HW), lambda b: (b, 0, 0)),
            full(C, 1), full(inner, 1), full(inner, C),
            full(inner, 1), full(inner, 1),
            full(inner, inner), full(2 * inner, inner),
            pl.BlockSpec((1, Lc, Dc), lambda b: (b, 0, 0)),
            full(2 * inner, Dc),
        ],
        out_specs=[
            pl.BlockSpec((1, inner, HW), lambda b: (b, 0, 0)),
            pl.BlockSpec((1, inner, HW), lambda b: (b, 0, 0)),
            pl.BlockSpec((1, 2 * inner, HW), lambda b: (b, 0, 0)),
            pl.BlockSpec((1, 2 * inner, Lc), lambda b: (b, 0, 0)),
        ],
        out_shape=[
            jax.ShapeDtypeStruct((B, inner, HW), _F32),
            jax.ShapeDtypeStruct((B, inner, HW), _BF),
            jax.ShapeDtypeStruct((B, 2 * inner, HW), _BF),
            jax.ShapeDtypeStruct((B, 2 * inner, Lc), _BF),
        ],
        compiler_params=sem,
    )(x3d, col(gn_gamma), beff, bt(w_in), col(g1), col(b1),
      qkv_t[:inner], qkv_t[inner:], context.astype(_BF), bt(a2_w_kv))

    tile = pl.BlockSpec((1, inner, tq), lambda b, qi: (b, 0, qi))
    x2res, q2 = pl.pallas_call(
        functools.partial(_attn1_kernel, heads=heads, dh=dh),
        grid=(B, QT),
        in_specs=[
            tile,
            pl.BlockSpec((1, 2 * inner, HW), lambda b, qi: (b, 0, 0)),
            full(inner, inner), full(inner, 1),
            tile,
            full(inner, 1), full(inner, 1),
            full(inner, inner),
        ],
        out_specs=[tile, tile],
        out_shape=[
            jax.ShapeDtypeStruct((B, inner, HW), _F32),
            jax.ShapeDtypeStruct((B, inner, HW), _BF),
        ],
        compiler_params=sem2,
    )(qt, kvt, bt(a1_w_o), col(a1_b_o), hres, col(g2), col(b2),
      bt(a2_w_q_scaled))

    out = pl.pallas_call(
        functools.partial(_attn2_ff_kernel, heads=heads, dh=dh),
        grid=(B, QT),
        in_specs=[
            tile,
            pl.BlockSpec((1, 2 * inner, Lc), lambda b, qi: (b, 0, 0)),
            full(inner, inner), full(inner, 1),
            tile,
            full(inner, 1), full(inner, 1),
            full(dff, inner), full(dff, inner), full(dff, 1), full(dff, 1),
            full(inner, dff), full(inner, 1),
            full(C, inner), full(C, 1),
            pl.BlockSpec((1, C, tq), lambda b, qi: (b, 0, qi)),
        ],
        out_specs=pl.BlockSpec((1, C, tq), lambda b, qi: (b, 0, qi)),
        out_shape=jax.ShapeDtypeStruct((B, C, HW), _F32),
        compiler_params=sem2,
    )(q2, kv2t, bt(a2_w_o), col(a2_b_o), x2res, col(g3), col(b3),
      bt(ff_w_x), bt(ff_w_g), col(ff_b_x), col(ff_b_g),
      bt(ff_w_o), col(ff_b_o), bt(w_out), col(b_out), x3d)

    return out.reshape(B, C, H, W)


# speedup vs baseline: 3.7516x; 1.1414x over previous
import functools

import jax
import jax.numpy as jnp
from jax.experimental import pallas as pl
from jax.experimental.pallas import tpu as pltpu

_VMEM_LIMIT = 64 * 1024 * 1024
_BF = jnp.bfloat16
_F32 = jnp.float32


def _ln_cm(x, g, b, eps=1e-5):
    mu = jnp.mean(x, axis=0, keepdims=True)
    xc = x - mu
    var = jnp.mean(xc * xc, axis=0, keepdims=True)
    return (xc * jax.lax.rsqrt(var + eps)) * g + b


def _mha_cm(q_loader, k_loader, v_loader, heads, dh):
    outs = []
    for h in range(heads):
        lo = h * dh
        qh = q_loader(lo)
        kh = k_loader(lo)
        vh = v_loader(lo)
        st = jax.lax.dot_general(kh, qh, (((0,), (0,)), ((), ())),
                                 preferred_element_type=_F32)
        m = jnp.max(st, axis=0, keepdims=True)
        p = jnp.exp(st - m)
        l = jnp.sum(p, axis=0, keepdims=True)
        ot = jnp.dot(vh, p.astype(_BF), preferred_element_type=_F32)
        outs.append(ot * (1.0 / l))
    return jnp.concatenate(outs, axis=0).astype(_BF)


def _pre_kernel(x_ref, gng_ref, beff_ref, w_in_t_ref, g1_ref, b1_ref,
                wq_t_ref, wkv_t_ref, ctx_ref, wkv2_t_ref,
                hres_ref, qt_ref, kvt_ref, kv2t_ref, *, groups):
    xg = x_ref[0].astype(_F32)
    C, HW = xg.shape
    xr = xg.reshape(groups, (C // groups) * HW)
    mu = jnp.mean(xr, axis=-1, keepdims=True)
    xc = xr - mu
    var = jnp.mean(xc * xc, axis=-1, keepdims=True)
    xn = (xc * jax.lax.rsqrt(var + 1e-6)).reshape(C, HW)
    xs = (xn * gng_ref[...].astype(_F32)).astype(_BF)
    h = jnp.dot(w_in_t_ref[...], xs, preferred_element_type=_F32)
    h = h + beff_ref[...]
    hres_ref[0] = h
    hn = _ln_cm(h, g1_ref[...], b1_ref[...]).astype(_BF)
    qt_ref[0] = jnp.dot(wq_t_ref[...], hn,
                        preferred_element_type=_F32).astype(_BF)
    kvt_ref[0] = jnp.dot(wkv_t_ref[...], hn,
                         preferred_element_type=_F32).astype(_BF)
    kv2t_ref[0] = jax.lax.dot_general(wkv2_t_ref[...], ctx_ref[0],
                                      (((1,), (1,)), ((), ())),
                                      preferred_element_type=_F32).astype(_BF)


def _attn1_kernel(qt_ref, kvt_ref, wo_t_ref, bo_ref, res_ref, g2_ref, b2_ref,
                  wq2_t_ref, x2_ref, q2_ref, *, heads, dh):
    inner = heads * dh
    attn = _mha_cm(
        lambda lo: qt_ref[0, lo:lo + dh, :],
        lambda lo: kvt_ref[0, lo:lo + dh, :],
        lambda lo: kvt_ref[0, inner + lo:inner + lo + dh, :],
        heads, dh)
    x2 = jnp.dot(wo_t_ref[...], attn, preferred_element_type=_F32)
    x2 = x2 + bo_ref[...] + res_ref[0]
    x2_ref[0] = x2
    hn = _ln_cm(x2, g2_ref[...], b2_ref[...]).astype(_BF)
    q2_ref[0] = jnp.dot(wq2_t_ref[...], hn,
                        preferred_element_type=_F32).astype(_BF)


def _attn2_ff_kernel(q2_ref, kv2t_ref, wo2_t_ref, bo2_ref, res_ref,
                     g3_ref, b3_ref, wx_t_ref, wg_t_ref, bx_ref, bg_ref,
                     wfo_t_ref, bfo_ref, wout_t_ref, bout_ref, xin_ref,
                     out_ref, *, heads, dh):
    inner = heads * dh
    attn = _mha_cm(
        lambda lo: q2_ref[0, lo:lo + dh, :],
        lambda lo: kv2t_ref[0, lo:lo + dh, :],
        lambda lo: kv2t_ref[0, inner + lo:inner + lo + dh, :],
        heads, dh)
    x3 = jnp.dot(wo2_t_ref[...], attn, preferred_element_type=_F32)
    x3 = x3 + bo2_ref[...] + res_ref[0]
    hn = _ln_cm(x3, g3_ref[...], b3_ref[...]).astype(_BF)
    u = jnp.dot(wx_t_ref[...], hn, preferred_element_type=_F32) + bx_ref[...]
    g = jnp.dot(wg_t_ref[...], hn, preferred_element_type=_F32) + bg_ref[...]
    gg = (u * jax.nn.gelu(g)).astype(_BF)
    x4 = jnp.dot(wfo_t_ref[...], gg, preferred_element_type=_F32)
    x4 = x4 + bfo_ref[...] + x3
    yt = jnp.dot(wout_t_ref[...], x4.astype(_BF),
                 preferred_element_type=_F32)
    out_ref[0] = yt + bout_ref[...] + xin_ref[0].astype(_F32)


def kernel(x, context, gn_gamma, gn_beta, w_in, b_in, w_out, b_out,
           g1, b1, g2, b2, g3, b3,
           a1_w_qkv, a1_w_q_scaled, a1_w_kv, a1_w_o, a1_b_o,
           a2_w_q_scaled, a2_w_kv, a2_w_o, a2_b_o,
           ff_w_x, ff_w_g, ff_b_x, ff_b_g, ff_w_o, ff_b_o):
    B, C, H, W = x.shape
    HW = H * W
    heads, dh = 8, 40
    inner = heads * dh
    Lc = context.shape[1]
    Dc = context.shape[2]
    dff = ff_w_x.shape[1]
    tq = min(1024, HW)
    QT = HW // tq

    x3d = x.reshape(B, C, HW)
    beff = (b_in + gn_beta @ w_in).reshape(-1, 1)
    col = lambda v: v.reshape(-1, 1)
    bt = lambda w: w.T.astype(_BF)
    qkv_t = a1_w_qkv.T.astype(_BF)

    sem = pltpu.CompilerParams(
        dimension_semantics=("parallel",), vmem_limit_bytes=_VMEM_LIMIT)
    sem2 = pltpu.CompilerParams(
        dimension_semantics=("parallel", "arbitrary"),
        vmem_limit_bytes=_VMEM_LIMIT)

    full = lambda *shape: pl.BlockSpec(shape, lambda b, qi=0: (0,) * len(shape))

    hres, qt, kvt, kv2t = pl.pallas_call(
        functools.partial(_pre_kernel, groups=32),
        grid=(B,),
        in_specs=[
            pl.BlockSpec((1, C, HW), lambda b: (b, 0, 0)),
            full(C, 1), full(inner, 1), full(inner, C),
            full(inner, 1), full(inner, 1),
            full(inner, inner), full(2 * inner, inner),
            pl.BlockSpec((1, Lc, Dc), lambda b: (b, 0, 0)),
            full(2 * inner, Dc),
        ],
        out_specs=[
            pl.BlockSpec((1, inner, HW), lambda b: (b, 0, 0)),
            pl.BlockSpec((1, inner, HW), lambda b: (b, 0, 0)),
            pl.BlockSpec((1, 2 * inner, HW), lambda b: (b, 0, 0)),
            pl.BlockSpec((1, 2 * inner, Lc), lambda b: (b, 0, 0)),
        ],
        out_shape=[
            jax.ShapeDtypeStruct((B, inner, HW), _F32),
            jax.ShapeDtypeStruct((B, inner, HW), _BF),
            jax.ShapeDtypeStruct((B, 2 * inner, HW), _BF),
            jax.ShapeDtypeStruct((B, 2 * inner, Lc), _BF),
        ],
        compiler_params=sem,
    )(x3d, col(gn_gamma), beff, bt(w_in), col(g1), col(b1),
      qkv_t[:inner], qkv_t[inner:], context.astype(_BF), bt(a2_w_kv))

    tile = pl.BlockSpec((1, inner, tq), lambda b, qi: (b, 0, qi))
    x2res, q2 = pl.pallas_call(
        functools.partial(_attn1_kernel, heads=heads, dh=dh),
        grid=(B, QT),
        in_specs=[
            tile,
            pl.BlockSpec((1, 2 * inner, HW), lambda b, qi: (b, 0, 0)),
            full(inner, inner), full(inner, 1),
            tile,
            full(inner, 1), full(inner, 1),
            full(inner, inner),
        ],
        out_specs=[tile, tile],
        out_shape=[
            jax.ShapeDtypeStruct((B, inner, HW), _F32),
            jax.ShapeDtypeStruct((B, inner, HW), _BF),
        ],
        compiler_params=sem2,
    )(qt, kvt, bt(a1_w_o), col(a1_b_o), hres, col(g2), col(b2),
      bt(a2_w_q_scaled))

    out = pl.pallas_call(
        functools.partial(_attn2_ff_kernel, heads=heads, dh=dh),
        grid=(B, QT),
        in_specs=[
            tile,
            pl.BlockSpec((1, 2 * inner, Lc), lambda b, qi: (b, 0, 0)),
            full(inner, inner), full(inner, 1),
            tile,
            full(inner, 1), full(inner, 1),
            full(dff, inner), full(dff, inner), full(dff, 1), full(dff, 1),
            full(inner, dff), full(inner, 1),
            full(C, inner), full(C, 1),
            pl.BlockSpec((1, C, tq), lambda b, qi: (b, 0, qi)),
        ],
        out_specs=pl.BlockSpec((1, C, tq), lambda b, qi: (b, 0, qi)),
        out_shape=jax.ShapeDtypeStruct((B, C, HW), _F32),
        compiler_params=sem2,
    )(q2, kv2t, bt(a2_w_o), col(a2_b_o), x2res, col(g3), col(b3),
      bt(ff_w_x), bt(ff_w_g), col(ff_b_x), col(ff_b_g),
      bt(ff_w_o), col(ff_b_o), bt(w_out), col(b_out), x3d)

    return out.reshape(B, C, H, W)


# speedup vs baseline: 3.9664x; 1.0572x over previous
import functools

import jax
import jax.numpy as jnp
from jax.experimental import pallas as pl
from jax.experimental.pallas import tpu as pltpu

_VMEM_LIMIT = 64 * 1024 * 1024
_BF = jnp.bfloat16
_F32 = jnp.float32


def _ln_cm(x, g, b, eps=1e-5):
    mu = jnp.mean(x, axis=0, keepdims=True)
    xc = x - mu
    var = jnp.mean(xc * xc, axis=0, keepdims=True)
    return (xc * jax.lax.rsqrt(var + eps)) * g + b


def _mha_cm(qt, kvt, heads, dh):
    inner = heads * dh
    outs = []
    for h in range(heads):
        lo = h * dh
        qh = qt[lo:lo + dh, :]
        kh = kvt[lo:lo + dh, :]
        vh = kvt[inner + lo:inner + lo + dh, :]
        st = jax.lax.dot_general(kh, qh, (((0,), (0,)), ((), ())),
                                 preferred_element_type=_F32)
        m = jnp.max(st, axis=0, keepdims=True)
        p = jnp.exp(st - m)
        l = jnp.sum(p, axis=0, keepdims=True)
        ot = jnp.dot(vh, p.astype(_BF), preferred_element_type=_F32)
        outs.append(ot * (1.0 / l))
    return jnp.concatenate(outs, axis=0).astype(_BF)


def _block_kernel(x_ref, ctx_ref, gng_ref, beff_ref, w_in_t_ref,
                  g1_ref, b1_ref, wq_t_ref, wkv_t_ref, wkv2_t_ref,
                  wo_t_ref, bo_ref, g2_ref, b2_ref, wq2_t_ref,
                  wo2_t_ref, bo2_ref, g3_ref, b3_ref,
                  wx_t_ref, wg_t_ref, bx_ref, bg_ref, wfo_t_ref, bfo_ref,
                  wout_t_ref, bout_ref, out_ref, *, groups, heads, dh):
    xg = x_ref[0].astype(_F32)
    C, HW = xg.shape
    xr = xg.reshape(groups, (C // groups) * HW)
    mu = jnp.mean(xr, axis=-1, keepdims=True)
    xc = xr - mu
    var = jnp.mean(xc * xc, axis=-1, keepdims=True)
    xn = (xc * jax.lax.rsqrt(var + 1e-6)).reshape(C, HW)
    xs = (xn * gng_ref[...].astype(_F32)).astype(_BF)
    h = jnp.dot(w_in_t_ref[...], xs, preferred_element_type=_F32)
    h = h + beff_ref[...]

    hn = _ln_cm(h, g1_ref[...], b1_ref[...]).astype(_BF)
    qt = jnp.dot(wq_t_ref[...], hn, preferred_element_type=_F32).astype(_BF)
    kvt = jnp.dot(wkv_t_ref[...], hn, preferred_element_type=_F32).astype(_BF)
    attn = _mha_cm(qt, kvt, heads, dh)
    x2 = jnp.dot(wo_t_ref[...], attn, preferred_element_type=_F32)
    x2 = x2 + bo_ref[...] + h

    kv2t = jax.lax.dot_general(wkv2_t_ref[...], ctx_ref[0],
                               (((1,), (1,)), ((), ())),
                               preferred_element_type=_F32).astype(_BF)
    hn2 = _ln_cm(x2, g2_ref[...], b2_ref[...]).astype(_BF)
    q2 = jnp.dot(wq2_t_ref[...], hn2, preferred_element_type=_F32).astype(_BF)
    attn2 = _mha_cm(q2, kv2t, heads, dh)
    x3 = jnp.dot(wo2_t_ref[...], attn2, preferred_element_type=_F32)
    x3 = x3 + bo2_ref[...] + x2

    hn3 = _ln_cm(x3, g3_ref[...], b3_ref[...]).astype(_BF)
    u = jnp.dot(wx_t_ref[...], hn3, preferred_element_type=_F32) + bx_ref[...]
    g = jnp.dot(wg_t_ref[...], hn3, preferred_element_type=_F32) + bg_ref[...]
    gg = (u * jax.nn.gelu(g)).astype(_BF)
    x4 = jnp.dot(wfo_t_ref[...], gg, preferred_element_type=_F32)
    x4 = x4 + bfo_ref[...] + x3

    yt = jnp.dot(wout_t_ref[...], x4.astype(_BF),
                 preferred_element_type=_F32)
    out_ref[0] = yt + bout_ref[...] + xg


def kernel(x, context, gn_gamma, gn_beta, w_in, b_in, w_out, b_out,
           g1, b1, g2, b2, g3, b3,
           a1_w_qkv, a1_w_q_scaled, a1_w_kv, a1_w_o, a1_b_o,
           a2_w_q_scaled, a2_w_kv, a2_w_o, a2_b_o,
           ff_w_x, ff_w_g, ff_b_x, ff_b_g, ff_w_o, ff_b_o):
    B, C, H, W = x.shape
    HW = H * W
    heads, dh = 8, 40
    inner = heads * dh
    Lc = context.shape[1]
    Dc = context.shape[2]
    dff = ff_w_x.shape[1]

    x3d = x.reshape(B, C, HW)
    beff = (b_in + gn_beta @ w_in).reshape(-1, 1)
    col = lambda v: v.reshape(-1, 1)
    bt = lambda w: w.T.astype(_BF)
    qkv_t = a1_w_qkv.T.astype(_BF)

    full = lambda *shape: pl.BlockSpec(shape, lambda b: (0,) * len(shape))

    out = pl.pallas_call(
        functools.partial(_block_kernel, groups=32, heads=heads, dh=dh),
        grid=(B,),
        in_specs=[
            pl.BlockSpec((1, C, HW), lambda b: (b, 0, 0)),
            pl.BlockSpec((1, Lc, Dc), lambda b: (b, 0, 0)),
            full(C, 1), full(inner, 1), full(inner, C),
            full(inner, 1), full(inner, 1),
            full(inner, inner), full(2 * inner, inner), full(2 * inner, Dc),
            full(inner, inner), full(inner, 1),
            full(inner, 1), full(inner, 1), full(inner, inner),
            full(inner, inner), full(inner, 1),
            full(inner, 1), full(inner, 1),
            full(dff, inner), full(dff, inner), full(dff, 1), full(dff, 1),
            full(inner, dff), full(inner, 1),
            full(C, inner), full(C, 1),
        ],
        out_specs=pl.BlockSpec((1, C, HW), lambda b: (b, 0, 0)),
        out_shape=jax.ShapeDtypeStruct((B, C, HW), _F32),
        compiler_params=pltpu.CompilerParams(
            dimension_semantics=("parallel",), vmem_limit_bytes=_VMEM_LIMIT),
    )(x3d, context.astype(_BF), col(gn_gamma), beff, bt(w_in),
      col(g1), col(b1), qkv_t[:inner], qkv_t[inner:], bt(a2_w_kv),
      bt(a1_w_o), col(a1_b_o), col(g2), col(b2), bt(a2_w_q_scaled),
      bt(a2_w_o), col(a2_b_o), col(g3), col(b3),
      bt(ff_w_x), bt(ff_w_g), col(ff_b_x), col(ff_b_g),
      bt(ff_w_o), col(ff_b_o), bt(w_out), col(b_out))

    return out.reshape(B, C, H, W)


# speedup vs baseline: 4.0412x; 1.0189x over previous
import functools

import jax
import jax.numpy as jnp
from jax.experimental import pallas as pl
from jax.experimental.pallas import tpu as pltpu

_VMEM_LIMIT = 64 * 1024 * 1024
_BF = jnp.bfloat16
_F32 = jnp.float32

_TA = (((0,), (0,)), ((), ()))


def _wdot(w, x):
    return jax.lax.dot_general(w, x, _TA, preferred_element_type=_F32)


def _ln_cm(x, g, b, eps=1e-5):
    mu = jnp.mean(x, axis=0, keepdims=True)
    xc = x - mu
    var = jnp.mean(xc * xc, axis=0, keepdims=True)
    return (xc * jax.lax.rsqrt(var + eps)) * g + b


def _mha_cm(qt, kvt, heads, dh):
    inner = heads * dh
    outs = []
    for h in range(heads):
        lo = h * dh
        qh = qt[lo:lo + dh, :]
        kh = kvt[lo:lo + dh, :]
        vh = kvt[inner + lo:inner + lo + dh, :]
        st = jax.lax.dot_general(kh, qh, _TA,
                                 preferred_element_type=_F32)
        m = jnp.max(st, axis=0, keepdims=True)
        p = jnp.exp(st - m)
        l = jnp.sum(p, axis=0, keepdims=True)
        ot = jnp.dot(vh, p.astype(_BF), preferred_element_type=_F32)
        outs.append(ot * (1.0 / l))
    return jnp.concatenate(outs, axis=0).astype(_BF)


def _block_kernel(x_ref, ctx_ref, gng_ref, beff_ref, w_in_ref,
                  g1_ref, b1_ref, wqkv_ref, wkv2_ref,
                  wo_ref, bo_ref, g2_ref, b2_ref, wq2_ref,
                  wo2_ref, bo2_ref, g3_ref, b3_ref,
                  wx_ref, wg_ref, bx_ref, bg_ref, wfo_ref, bfo_ref,
                  wout_ref, bout_ref, out_ref, *, groups, heads, dh):
    inner = heads * dh
    xg = x_ref[0].astype(_F32)
    C, HW = xg.shape
    xr = xg.reshape(groups, (C // groups) * HW)
    mu = jnp.mean(xr, axis=-1, keepdims=True)
    xc = xr - mu
    var = jnp.mean(xc * xc, axis=-1, keepdims=True)
    xn = (xc * jax.lax.rsqrt(var + 1e-6)).reshape(C, HW)
    xs = (xn * gng_ref[...].astype(_F32)).astype(_BF)
    h = _wdot(w_in_ref[...], xs) + beff_ref[...]

    hn = _ln_cm(h, g1_ref[...], b1_ref[...]).astype(_BF)
    qkv = _wdot(wqkv_ref[...], hn).astype(_BF)
    attn = _mha_cm(qkv[:inner], qkv[inner:], heads, dh)
    x2 = _wdot(wo_ref[...], attn) + bo_ref[...] + h

    kv2t = jax.lax.dot_general(wkv2_ref[...], ctx_ref[0],
                               (((0,), (1,)), ((), ())),
                               preferred_element_type=_F32).astype(_BF)
    hn2 = _ln_cm(x2, g2_ref[...], b2_ref[...]).astype(_BF)
    q2 = _wdot(wq2_ref[...], hn2).astype(_BF)
    attn2 = _mha_cm(q2, kv2t, heads, dh)
    x3 = _wdot(wo2_ref[...], attn2) + bo2_ref[...] + x2

    hn3 = _ln_cm(x3, g3_ref[...], b3_ref[...]).astype(_BF)
    u = _wdot(wx_ref[...], hn3) + bx_ref[...]
    g = _wdot(wg_ref[...], hn3) + bg_ref[...]
    gg = (u * jax.nn.gelu(g)).astype(_BF)
    x4 = _wdot(wfo_ref[...], gg) + bfo_ref[...] + x3

    yt = _wdot(wout_ref[...], x4.astype(_BF))
    out_ref[0] = yt + bout_ref[...] + xg


def kernel(x, context, gn_gamma, gn_beta, w_in, b_in, w_out, b_out,
           g1, b1, g2, b2, g3, b3,
           a1_w_qkv, a1_w_q_scaled, a1_w_kv, a1_w_o, a1_b_o,
           a2_w_q_scaled, a2_w_kv, a2_w_o, a2_b_o,
           ff_w_x, ff_w_g, ff_b_x, ff_b_g, ff_w_o, ff_b_o):
    B, C, H, W = x.shape
    HW = H * W
    heads, dh = 8, 40
    inner = heads * dh
    Lc = context.shape[1]
    Dc = context.shape[2]
    dff = ff_w_x.shape[1]

    x3d = x.reshape(B, C, HW)
    beff = (b_in + gn_beta @ w_in).reshape(-1, 1)
    col = lambda v: v.reshape(-1, 1)
    bf = lambda w: w.astype(_BF)

    full = lambda *shape: pl.BlockSpec(shape, lambda b: (0,) * len(shape))

    out = pl.pallas_call(
        functools.partial(_block_kernel, groups=32, heads=heads, dh=dh),
        grid=(B,),
        in_specs=[
            pl.BlockSpec((1, C, HW), lambda b: (b, 0, 0)),
            pl.BlockSpec((1, Lc, Dc), lambda b: (b, 0, 0)),
            full(C, 1), full(inner, 1), full(C, inner),
            full(inner, 1), full(inner, 1),
            full(inner, 3 * inner), full(Dc, 2 * inner),
            full(inner, inner), full(inner, 1),
            full(inner, 1), full(inner, 1), full(inner, inner),
            full(inner, inner), full(inner, 1),
            full(inner, 1), full(inner, 1),
            full(inner, dff), full(inner, dff), full(dff, 1), full(dff, 1),
            full(dff, inner), full(inner, 1),
            full(inner, C), full(C, 1),
        ],
        out_specs=pl.BlockSpec((1, C, HW), lambda b: (b, 0, 0)),
        out_shape=jax.ShapeDtypeStruct((B, C, HW), _F32),
        compiler_params=pltpu.CompilerParams(
            dimension_semantics=("parallel",), vmem_limit_bytes=_VMEM_LIMIT),
    )(x3d, context.astype(_BF), col(gn_gamma), beff, bf(w_in),
      col(g1), col(b1), bf(a1_w_qkv), bf(a2_w_kv),
      bf(a1_w_o), col(a1_b_o), col(g2), col(b2), bf(a2_w_q_scaled),
      bf(a2_w_o), col(a2_b_o), col(g3), col(b3),
      bf(ff_w_x), bf(ff_w_g), col(ff_b_x), col(ff_b_g),
      bf(ff_w_o), col(ff_b_o), bf(w_out), col(b_out))

    return out.reshape(B, C, H, W)
